# baseline (speedup 1.0000x reference)
import functools

import jax
import jax.numpy as jnp
from jax import lax
from jax.experimental import pallas as pl
from jax.experimental.pallas import tpu as pltpu
from jax.experimental.pallas import tpu_sc as plsc

_CENTER_VAR = 0.1
_SIZE_VAR = 0.2
_NUM_CLASSES = 21
_TOP_K = 200
_IOU_THR = 0.45
_SCORE_THR = 0.01

_B = 16
_N = 20000
_NB = 1024
_NBLK = 20
_NPAD = _NB * _NBLK
_ROWS = 160
_LANES = 128
_KPAD = 256
_CBUF = 320
_HALF = _NPAD // 2
_BIG = 2 ** 30
_NEG = float("-inf")


def _decode_body(logits_ref, bbox_ref, priors_ref, packed_ref, best_ref):
    nb = pl.program_id(1)
    xt = logits_ref[0]
    m = jnp.max(xt, axis=0, keepdims=True)
    e = jnp.exp(xt - m)
    s = jnp.sum(e, axis=0, keepdims=True)
    efg = e[1:, :]
    eb = jnp.max(efg, axis=0, keepdims=True)
    best = eb / s
    cls_iota = lax.broadcasted_iota(jnp.int32, efg.shape, 0)
    lab = jnp.min(jnp.where(efg >= eb, cls_iota, _BIG), axis=0, keepdims=True) + 1

    lane_iota = lax.broadcasted_iota(jnp.int32, (1, _NB), 1)
    valid = (nb * _NB + lane_iota) < _N
    best = jnp.where(valid, best, _NEG)
    labf = jnp.where(valid, lab, 0).astype(jnp.float32)

    bt = bbox_ref[0]
    pt = priors_ref[...]
    cx = bt[0:1] * _CENTER_VAR * pt[2:3] + pt[0:1]
    cy = bt[1:2] * _CENTER_VAR * pt[3:4] + pt[1:2]
    w = jnp.exp(bt[2:3] * _SIZE_VAR) * pt[2:3]
    h = jnp.exp(bt[3:4] * _SIZE_VAR) * pt[3:4]
    x1 = cx - w * 0.5
    y1 = cy - h * 0.5
    x2 = cx + w * 0.5
    y2 = cy + h * 0.5
    z = jnp.zeros((1, _NB), jnp.float32)
    stack = jnp.concatenate([best, labf, x1, y1, x2, y2, z, z], axis=0)
    packed_ref[0] = jnp.transpose(stack, (1, 0))
    best_ref[0, 0] = best


def _bisect_body(best_ref, thr_ref):
    view = lax.bitcast_convert_type(best_ref[...], jnp.int32)

    def body(_, carry):
        lo, hi = carry
        mid = lo + (hi - lo) // 2
        ge = (view >= mid).astype(jnp.int32)
        cnt = jnp.sum(jnp.sum(ge, axis=1, keepdims=True), axis=2, keepdims=True)
        take = cnt >= _TOP_K
        return jnp.where(take, mid, lo), jnp.where(take, hi, mid)

    lo0 = jnp.zeros((_B, 1, 1), jnp.int32)
    hi0 = jnp.full((_B, 1, 1), 2 ** 31 - 1, jnp.int32)
    lo, _ = lax.fori_loop(0, 31, body, (lo0, hi0))
    thr = lax.bitcast_convert_type(lo, jnp.float32)
    thr_ref[...] = jnp.broadcast_to(thr, (_B, 1, 16))


def _sc_compact_body(best_hbm, thr_hbm, packed_hbm, vals_hbm, idx_hbm, rows_hbm,
                     chunk_v, sv_v, si_v, thr_v, packed_v, srows_v):
    cid = lax.axis_index("c")
    sid = lax.axis_index("s")
    wid = sid * 2 + cid
    b = wid // 2
    h = wid - b * 2
    base0 = h * _HALF

    pltpu.sync_copy(best_hbm.at[b, pl.ds(base0, _HALF)], chunk_v)
    pltpu.sync_copy(packed_hbm.at[b, pl.ds(base0, _HALF)], packed_v)
    pltpu.sync_copy(thr_hbm.at[b], thr_v)
    t16 = thr_v[...]

    neg = jnp.full((16,), _NEG, jnp.float32)
    zero = jnp.zeros((16,), jnp.int32)

    def init(j, _):
        sv_v[pl.ds(j * 16, 16)] = neg
        si_v[pl.ds(j * 16, 16)] = zero + base0
        return 0

    lax.fori_loop(0, _CBUF // 16, init, 0)

    lane = lax.iota(jnp.int32, 16)
    one16 = jnp.ones((16,), jnp.int32)
    zero16 = jnp.zeros((16,), jnp.int32)

    def body(i, off):
        v = chunk_v[pl.ds(i * 16, 16)]
        mask = v >= t16
        cnt = jnp.sum(jnp.where(mask, one16, zero16))
        off_c = jnp.minimum(off, _CBUF - 16)
        idxv = base0 + i * 16 + lane
        plsc.store_compressed(sv_v.at[pl.ds(off_c, 16)], v, mask=mask)
        plsc.store_compressed(si_v.at[pl.ds(off_c, 16)], idxv, mask=mask)
        return jnp.minimum(off + cnt, _CBUF - 16)

    lax.fori_loop(0, _HALF // 16, body, jnp.int32(0))

    def ggroup(g, _):
        rows16 = si_v[pl.ds(g * 16, 16)] - base0
        dst16 = g * 16 + lane
        for f in range(8):
            fs = zero + f
            v = plsc.load_gather(packed_v, [rows16, fs])
            plsc.store_scatter(srows_v, [dst16, fs], v)
        return 0

    lax.fori_loop(0, _CBUF // 16, ggroup, 0)

    pltpu.sync_copy(sv_v, vals_hbm.at[b, h])
    pltpu.sync_copy(si_v, idx_hbm.at[b, h])
    pltpu.sync_copy(srows_v, rows_hbm.at[b, h])


_NSRV = 2 * _CBUF


def _nms_body(srows_ref, sv_ref, si_ref,
              boxes_ref, scores_ref, labels_ref, keep_ref,
              cand_rows, supmat):
    lane640 = lax.broadcasted_iota(jnp.int32, (_B, 1, _NSRV), 2)
    lane256 = lax.broadcasted_iota(jnp.int32, (_B, 1, _KPAD), 2)
    si = si_ref[...]

    def sort_step(k, carry):
        svv, va, rank = carry
        m = jnp.max(svv, axis=2, keepdims=True)
        idx = jnp.min(jnp.where(svv == m, si, _BIG), axis=2, keepdims=True)
        hit = si == idx
        pos = jnp.min(jnp.where(hit, lane640, _BIG), axis=2, keepdims=True)
        va = jnp.where(lane256 == k, jnp.broadcast_to(m, va.shape), va)
        rank = jnp.where(lane640 == pos, k, rank)
        svv = jnp.where(hit, _NEG, svv)
        return svv, va, rank

    va0 = jnp.full((_B, 1, _KPAD), _NEG, jnp.float32)
    rank0 = jnp.full((_B, 1, _NSRV), _BIG, jnp.int32)
    _, va, rank = lax.fori_loop(0, _TOP_K, sort_step,
                                (sv_ref[...], va0, rank0))

    def rows_step(j, carry):
        x1r, y1r, x2r, y2r, labr = carry
        row = srows_ref[:, pl.ds(j, 1), :]
        r = jnp.min(jnp.where(lane640 == j, rank, _BIG), axis=2, keepdims=True)
        sel = lane256 == r

        def put(acc, f):
            v = row[:, :, f:f + 1]
            return jnp.where(sel, jnp.broadcast_to(v, acc.shape), acc)

        return (put(x1r, 2), put(y1r, 3), put(x2r, 4), put(y2r, 5),
                put(labr, 1))

    z = jnp.zeros((_B, 1, _KPAD), jnp.float32)
    x1r, y1r, x2r, y2r, labr = lax.fori_loop(
        0, _NSRV, rows_step, (z, z, z, z, z))

    def cols_step(k, _):
        sel = lane256 == k
        vx1 = jnp.max(jnp.where(sel, x1r, _NEG), axis=2, keepdims=True)
        vy1 = jnp.max(jnp.where(sel, y1r, _NEG), axis=2, keepdims=True)
        vx2 = jnp.max(jnp.where(sel, x2r, _NEG), axis=2, keepdims=True)
        vy2 = jnp.max(jnp.where(sel, y2r, _NEG), axis=2, keepdims=True)
        vlb = jnp.max(jnp.where(sel, labr, _NEG), axis=2, keepdims=True)
        cand_rows[:, pl.ds(k, 1), 0:5] = jnp.concatenate(
            [vx1, vy1, vx2, vy2, vlb], axis=2)
        return 0

    lax.fori_loop(0, _TOP_K, cols_step, 0)

    x1col = cand_rows[:, :, 0:1]
    y1col = cand_rows[:, :, 1:2]
    x2col = cand_rows[:, :, 2:3]
    y2col = cand_rows[:, :, 3:4]
    labcol = cand_rows[:, :, 4:5]

    ix1 = jnp.maximum(x1col, x1r)
    iy1 = jnp.maximum(y1col, y1r)
    ix2 = jnp.minimum(x2col, x2r)
    iy2 = jnp.minimum(y2col, y2r)
    inter = jnp.clip(ix2 - ix1, 0.0) * jnp.clip(iy2 - iy1, 0.0)
    area_col = jnp.clip(x2col - x1col, 0.0) * jnp.clip(y2col - y1col, 0.0)
    area_row = jnp.clip(x2r - x1r, 0.0) * jnp.clip(y2r - y1r, 0.0)
    union = area_col + area_row - inter
    iou = inter / jnp.maximum(union, 1e-9)
    same = labcol == labr
    supmat[...] = jnp.where((iou > _IOU_THR) & same, 1.0, 0.0)

    keep0 = jnp.where(va > _SCORE_THR, 1.0, 0.0)

    def nms_step(i, keepf):
        rowi = supmat[:, pl.ds(i, 1), :]
        cansup = rowi * keepf * jnp.where(lane256 < i, 1.0, 0.0)
        supp = jnp.max(cansup, axis=2, keepdims=True)
        return jnp.where(lane256 == i,
                         keepf * (1.0 - jnp.broadcast_to(supp, keepf.shape)),
                         keepf)

    keepf = lax.fori_loop(0, _TOP_K, nms_step, keep0)

    scores_ref[...] = (va * keepf)[:, :, :_TOP_K]
    labels_ref[...] = (labr * keepf)[:, :, :_TOP_K].astype(jnp.int32)
    keep_ref[...] = keepf[:, :, :_TOP_K].astype(jnp.int32)

    def out_body(k, _):
        kv = jnp.max(jnp.where(lane256 == k, keepf, 0.0), axis=2, keepdims=True)
        row = cand_rows[:, pl.ds(k, 1), 0:4] * jnp.broadcast_to(kv, (_B, 1, 4))
        boxes_ref[:, pl.ds(k, 1), :] = row
        return 0

    lax.fori_loop(0, _TOP_K, out_body, 0)


def _sc_compact():
    mesh = plsc.VectorSubcoreMesh(core_axis_name="c", subcore_axis_name="s")
    return pl.kernel(
        _sc_compact_body, mesh=mesh,
        compiler_params=pltpu.CompilerParams(needs_layout_passes=False,
                                             use_tc_tiling_on_sc=False),
        out_type=[jax.ShapeDtypeStruct((_B, 2, _CBUF), jnp.float32),
                  jax.ShapeDtypeStruct((_B, 2, _CBUF), jnp.int32),
                  jax.ShapeDtypeStruct((_B, 2, _CBUF, 8), jnp.float32)],
        scratch_types=[pltpu.VMEM((_HALF,), jnp.float32),
                       pltpu.VMEM((_CBUF,), jnp.float32),
                       pltpu.VMEM((_CBUF,), jnp.int32),
                       pltpu.VMEM((16,), jnp.float32),
                       pltpu.VMEM((_HALF, 8), jnp.float32),
                       pltpu.VMEM((_CBUF, 8), jnp.float32)],
    )


def kernel(cls_logits, bbox_pred, priors):
    B, N, C = cls_logits.shape

    logits_t = jnp.swapaxes(cls_logits, 1, 2)
    bbox_t = jnp.swapaxes(bbox_pred, 1, 2)
    priors_t = jnp.transpose(priors, (1, 0))

    packed, bestp = pl.pallas_call(
        _decode_body,
        grid=(B, _NBLK),
        in_specs=[
            pl.BlockSpec((1, _NUM_CLASSES, _NB), lambda b, nb: (b, 0, nb)),
            pl.BlockSpec((1, 4, _NB), lambda b, nb: (b, 0, nb)),
            pl.BlockSpec((4, _NB), lambda b, nb: (0, nb)),
        ],
        out_specs=[
            pl.BlockSpec((1, _NB, 8), lambda b, nb: (b, nb, 0)),
            pl.BlockSpec((1, 1, 1, _NB), lambda b, nb: (b, nb, 0, 0)),
        ],
        out_shape=[
            jax.ShapeDtypeStruct((B, _NPAD, 8), jnp.float32),
            jax.ShapeDtypeStruct((B, _NBLK, 1, _NB), jnp.float32),
        ],
    )(logits_t, bbox_t, priors_t)

    best2d = bestp.reshape(B, _ROWS, _LANES)

    thr = pl.pallas_call(
        _bisect_body,
        in_specs=[pl.BlockSpec((B, _ROWS, _LANES), lambda: (0, 0, 0))],
        out_specs=pl.BlockSpec((B, 1, 16), lambda: (0, 0, 0)),
        out_shape=jax.ShapeDtypeStruct((B, 1, 16), jnp.float32),
    )(best2d)

    sv, si, srows = _sc_compact()(best2d.reshape(B, _NPAD),
                                  thr.reshape(B, 16), packed)

    boxes, scores, labels, keep = pl.pallas_call(
        _nms_body,
        in_specs=[
            pl.BlockSpec((B, _NSRV, 8), lambda: (0, 0, 0)),
            pl.BlockSpec((B, 1, _NSRV), lambda: (0, 0, 0)),
            pl.BlockSpec((B, 1, _NSRV), lambda: (0, 0, 0)),
        ],
        out_specs=[
            pl.BlockSpec((B, _TOP_K, 4), lambda: (0, 0, 0)),
            pl.BlockSpec((B, 1, _TOP_K), lambda: (0, 0, 0)),
            pl.BlockSpec((B, 1, _TOP_K), lambda: (0, 0, 0)),
            pl.BlockSpec((B, 1, _TOP_K), lambda: (0, 0, 0)),
        ],
        out_shape=[
            jax.ShapeDtypeStruct((B, _TOP_K, 4), jnp.float32),
            jax.ShapeDtypeStruct((B, 1, _TOP_K), jnp.float32),
            jax.ShapeDtypeStruct((B, 1, _TOP_K), jnp.int32),
            jax.ShapeDtypeStruct((B, 1, _TOP_K), jnp.int32),
        ],
        scratch_shapes=[
            pltpu.VMEM((_B, _KPAD, 8), jnp.float32),
            pltpu.VMEM((_B, _KPAD, _KPAD), jnp.float32),
        ],
    )(srows.reshape(B, _NSRV, 8), sv.reshape(B, 1, _NSRV),
      si.reshape(B, 1, _NSRV))

    return (boxes,
            scores.reshape(B, _TOP_K),
            labels.reshape(B, _TOP_K),
            keep.reshape(B, _TOP_K).astype(bool))


# speedup vs baseline: 1.3551x; 1.3551x over previous
import functools

import jax
import jax.numpy as jnp
from jax import lax
from jax.experimental import pallas as pl
from jax.experimental.pallas import tpu as pltpu
from jax.experimental.pallas import tpu_sc as plsc

_CENTER_VAR = 0.1
_SIZE_VAR = 0.2
_NUM_CLASSES = 21
_TOP_K = 200
_IOU_THR = 0.45
_SCORE_THR = 0.01

_B = 16
_N = 20000
_NB = 1024
_NBLK = 20
_NPAD = _NB * _NBLK
_ROWS = 160
_LANES = 128
_KPAD = 256
_CBUF = 320
_HALF = _NPAD // 2
_BIG = 2 ** 30
_NEG = float("-inf")


def _decode_body(logits_ref, bbox_ref, priors_ref, packed_ref, best_ref):
    nb = pl.program_id(1)
    xt = logits_ref[0]
    m = jnp.max(xt, axis=0, keepdims=True)
    e = jnp.exp(xt - m)
    s = jnp.sum(e, axis=0, keepdims=True)
    efg = e[1:, :]
    eb = jnp.max(efg, axis=0, keepdims=True)
    best = eb / s
    cls_iota = lax.broadcasted_iota(jnp.int32, efg.shape, 0)
    lab = jnp.min(jnp.where(efg >= eb, cls_iota, _BIG), axis=0, keepdims=True) + 1

    lane_iota = lax.broadcasted_iota(jnp.int32, (1, _NB), 1)
    valid = (nb * _NB + lane_iota) < _N
    best = jnp.where(valid, best, _NEG)
    labf = jnp.where(valid, lab, 0).astype(jnp.float32)

    bt = bbox_ref[0]
    pt = priors_ref[...]
    cx = bt[0:1] * _CENTER_VAR * pt[2:3] + pt[0:1]
    cy = bt[1:2] * _CENTER_VAR * pt[3:4] + pt[1:2]
    w = jnp.exp(bt[2:3] * _SIZE_VAR) * pt[2:3]
    h = jnp.exp(bt[3:4] * _SIZE_VAR) * pt[3:4]
    x1 = cx - w * 0.5
    y1 = cy - h * 0.5
    x2 = cx + w * 0.5
    y2 = cy + h * 0.5
    z = jnp.zeros((1, _NB), jnp.float32)
    stack = jnp.concatenate([best, labf, x1, y1, x2, y2, z, z], axis=0)
    packed_ref[0] = jnp.transpose(stack, (1, 0))
    best_ref[0, 0] = best


def _bisect_body(best_ref, thr_ref):
    view = lax.bitcast_convert_type(best_ref[...], jnp.int32)

    def body(_, carry):
        lo, hi = carry
        mid = lo + (hi - lo) // 2
        ge = (view >= mid).astype(jnp.int32)
        cnt = jnp.sum(jnp.sum(ge, axis=1, keepdims=True), axis=2, keepdims=True)
        take = cnt >= _TOP_K
        return jnp.where(take, mid, lo), jnp.where(take, hi, mid)

    lo0 = jnp.zeros((_B, 1, 1), jnp.int32)
    hi0 = jnp.full((_B, 1, 1), 2 ** 31 - 1, jnp.int32)
    lo, _ = lax.fori_loop(0, 31, body, (lo0, hi0))
    thr = lax.bitcast_convert_type(lo, jnp.float32)
    thr_ref[...] = jnp.broadcast_to(thr, (_B, 1, 16))


def _sc_compact_body(best_hbm, thr_hbm, vals_hbm, idx_hbm,
                     chunk_v, sv_v, si_v, thr_v):
    cid = lax.axis_index("c")
    sid = lax.axis_index("s")
    wid = sid * 2 + cid
    b = wid // 2
    h = wid - b * 2

    pltpu.sync_copy(best_hbm.at[b, pl.ds(h * _HALF, _HALF)], chunk_v)
    pltpu.sync_copy(thr_hbm.at[b], thr_v)
    t16 = thr_v[...]

    neg = jnp.full((16,), _NEG, jnp.float32)
    zero = jnp.zeros((16,), jnp.int32)

    def init(j, _):
        sv_v[pl.ds(j * 16, 16)] = neg
        si_v[pl.ds(j * 16, 16)] = zero
        return 0

    lax.fori_loop(0, _CBUF // 16, init, 0)

    lane = lax.iota(jnp.int32, 16)
    base0 = h * _HALF

    one16 = jnp.ones((16,), jnp.int32)
    zero16 = jnp.zeros((16,), jnp.int32)

    def body(i, off):
        v = chunk_v[pl.ds(i * 16, 16)]
        mask = v >= t16
        cnt = jnp.sum(jnp.where(mask, one16, zero16))
        off_c = jnp.minimum(off, _CBUF - 16)
        idxv = base0 + i * 16 + lane
        plsc.store_compressed(sv_v.at[pl.ds(off_c, 16)], v, mask=mask)
        plsc.store_compressed(si_v.at[pl.ds(off_c, 16)], idxv, mask=mask)
        return jnp.minimum(off + cnt, _CBUF - 16)

    lax.fori_loop(0, _HALF // 16, body, jnp.int32(0))

    pltpu.sync_copy(sv_v, vals_hbm.at[b, h])
    pltpu.sync_copy(si_v, idx_hbm.at[b, h])


def _sort_body(sv_ref, si_ref, gv_ref, gi_ref):
    sv = sv_ref[...]
    si = si_ref[...]
    kiota = lax.broadcasted_iota(jnp.int32, (_B, 1, _KPAD), 2)
    biota = lax.broadcasted_iota(jnp.int32, (_B, 1, _KPAD), 0)

    def body(k, carry):
        sv, va, ia = carry
        m = jnp.max(jnp.max(sv, axis=1, keepdims=True), axis=2, keepdims=True)
        sel = jnp.where(sv == m, si, _BIG)
        idx = jnp.min(jnp.min(sel, axis=1, keepdims=True), axis=2, keepdims=True)
        va = jnp.where(kiota == k, jnp.broadcast_to(m, va.shape), va)
        ia = jnp.where(kiota == k, jnp.broadcast_to(idx, ia.shape), ia)
        sv = jnp.where(si == idx, _NEG, sv)
        return sv, va, ia

    va0 = jnp.full((_B, 1, _KPAD), _NEG, jnp.float32)
    ia0 = jnp.zeros((_B, 1, _KPAD), jnp.int32)
    _, va, ia = lax.fori_loop(0, _TOP_K, body, (sv, va0, ia0))
    gv_ref[...] = va
    gi_ref[...] = ia + biota * _NPAD


def _sc_gather_body(packed_hbm, gidx_hbm, cand_hbm, idx_v, rows_v, sem):
    cid = lax.axis_index("c")
    sid = lax.axis_index("s")
    wid = sid * 2 + cid
    b = wid // 2
    h = wid - b * 2

    pltpu.sync_copy(gidx_hbm.at[b, pl.ds(h * 128, 128)], idx_v)
    pltpu.async_copy(packed_hbm.at[idx_v], rows_v, sem).wait()
    pltpu.sync_copy(rows_v, cand_hbm.at[b, pl.ds(h * 128, 128)])


def _nms_body(cand_ref, gv_ref, boxes_ref, scores_ref, labels_ref, keep_ref,
              supmat):
    lane256 = lax.broadcasted_iota(jnp.int32, (_B, 1, _KPAD), 2)
    cand = cand_ref[...]
    labcol = cand[:, :, 1:2]
    x1col = cand[:, :, 2:3]
    y1col = cand[:, :, 3:4]
    x2col = cand[:, :, 4:5]
    y2col = cand[:, :, 5:6]

    def gather_rows(k, carry):
        x1r, y1r, x2r, y2r, labr = carry
        row = cand_ref[:, pl.ds(k, 1), :]
        sel = lane256 == k

        def put(acc, f):
            v = row[:, :, f:f + 1]
            return jnp.where(sel, jnp.broadcast_to(v, acc.shape), acc)

        return (put(x1r, 2), put(y1r, 3), put(x2r, 4), put(y2r, 5),
                put(labr, 1))

    z = jnp.zeros((_B, 1, _KPAD), jnp.float32)
    x1r, y1r, x2r, y2r, labr = lax.fori_loop(
        0, _TOP_K, gather_rows, (z, z, z, z, z))

    ix1 = jnp.maximum(x1col, x1r)
    iy1 = jnp.maximum(y1col, y1r)
    ix2 = jnp.minimum(x2col, x2r)
    iy2 = jnp.minimum(y2col, y2r)
    inter = jnp.clip(ix2 - ix1, 0.0) * jnp.clip(iy2 - iy1, 0.0)
    area_col = jnp.clip(x2col - x1col, 0.0) * jnp.clip(y2col - y1col, 0.0)
    area_row = jnp.clip(x2r - x1r, 0.0) * jnp.clip(y2r - y1r, 0.0)
    union = area_col + area_row - inter
    iou = inter / jnp.maximum(union, 1e-9)
    same = labcol == labr
    supmat[...] = jnp.where((iou > _IOU_THR) & same, 1.0, 0.0)

    gv = gv_ref[...]
    keep0 = jnp.where(gv > _SCORE_THR, 1.0, 0.0)

    def nms_step(i, keepf):
        rowi = supmat[:, pl.ds(i, 1), :]
        cansup = rowi * keepf * jnp.where(lane256 < i, 1.0, 0.0)
        supp = jnp.max(cansup, axis=2, keepdims=True)
        return jnp.where(lane256 == i,
                         keepf * (1.0 - jnp.broadcast_to(supp, keepf.shape)),
                         keepf)

    keepf = lax.fori_loop(0, _TOP_K, nms_step, keep0)

    scores_ref[...] = (gv * keepf)[:, :, :_TOP_K]
    labels_ref[...] = (labr * keepf)[:, :, :_TOP_K].astype(jnp.int32)
    keep_ref[...] = keepf[:, :, :_TOP_K].astype(jnp.int32)

    def out_body(k, _):
        kv = jnp.max(jnp.where(lane256 == k, keepf, 0.0), axis=2, keepdims=True)
        row = cand_ref[:, pl.ds(k, 1), 2:6] * jnp.broadcast_to(kv, (_B, 1, 4))
        boxes_ref[:, pl.ds(k, 1), :] = row
        return 0

    lax.fori_loop(0, _TOP_K, out_body, 0)


def _sc_compact():
    mesh = plsc.VectorSubcoreMesh(core_axis_name="c", subcore_axis_name="s")
    return pl.kernel(
        _sc_compact_body, mesh=mesh,
        compiler_params=pltpu.CompilerParams(needs_layout_passes=False),
        out_type=[jax.ShapeDtypeStruct((_B, 2, _CBUF), jnp.float32),
                  jax.ShapeDtypeStruct((_B, 2, _CBUF), jnp.int32)],
        scratch_types=[pltpu.VMEM((_HALF,), jnp.float32),
                       pltpu.VMEM((_CBUF,), jnp.float32),
                       pltpu.VMEM((_CBUF,), jnp.int32),
                       pltpu.VMEM((16,), jnp.float32)],
    )


def _sc_gather():
    mesh = plsc.VectorSubcoreMesh(core_axis_name="c", subcore_axis_name="s")
    return pl.kernel(
        _sc_gather_body, mesh=mesh,
        compiler_params=pltpu.CompilerParams(needs_layout_passes=False,
                                             use_tc_tiling_on_sc=False),
        out_type=jax.ShapeDtypeStruct((_B, _KPAD, 8), jnp.float32),
        scratch_types=[pltpu.VMEM((128,), jnp.int32),
                       pltpu.VMEM((128, 8), jnp.float32),
                       pltpu.SemaphoreType.DMA],
    )


def kernel(cls_logits, bbox_pred, priors):
    B, N, C = cls_logits.shape

    logits_t = jnp.swapaxes(cls_logits, 1, 2)
    bbox_t = jnp.swapaxes(bbox_pred, 1, 2)
    priors_t = jnp.transpose(priors, (1, 0))

    packed, bestp = pl.pallas_call(
        _decode_body,
        grid=(B, _NBLK),
        in_specs=[
            pl.BlockSpec((1, _NUM_CLASSES, _NB), lambda b, nb: (b, 0, nb)),
            pl.BlockSpec((1, 4, _NB), lambda b, nb: (b, 0, nb)),
            pl.BlockSpec((4, _NB), lambda b, nb: (0, nb)),
        ],
        out_specs=[
            pl.BlockSpec((1, _NB, 8), lambda b, nb: (b, nb, 0)),
            pl.BlockSpec((1, 1, 1, _NB), lambda b, nb: (b, nb, 0, 0)),
        ],
        out_shape=[
            jax.ShapeDtypeStruct((B, _NPAD, 8), jnp.float32),
            jax.ShapeDtypeStruct((B, _NBLK, 1, _NB), jnp.float32),
        ],
    )(logits_t, bbox_t, priors_t)

    best2d = bestp.reshape(B, _ROWS, _LANES)

    thr = pl.pallas_call(
        _bisect_body,
        in_specs=[pl.BlockSpec((B, _ROWS, _LANES), lambda: (0, 0, 0))],
        out_specs=pl.BlockSpec((B, 1, 16), lambda: (0, 0, 0)),
        out_shape=jax.ShapeDtypeStruct((B, 1, 16), jnp.float32),
    )(best2d)

    sv, si = _sc_compact()(best2d.reshape(B, _NPAD), thr.reshape(B, 16))

    gv, gi = pl.pallas_call(
        _sort_body,
        in_specs=[
            pl.BlockSpec((B, 1, 2 * _CBUF), lambda: (0, 0, 0)),
            pl.BlockSpec((B, 1, 2 * _CBUF), lambda: (0, 0, 0)),
        ],
        out_specs=[
            pl.BlockSpec((B, 1, _KPAD), lambda: (0, 0, 0)),
            pl.BlockSpec((B, 1, _KPAD), lambda: (0, 0, 0)),
        ],
        out_shape=[
            jax.ShapeDtypeStruct((B, 1, _KPAD), jnp.float32),
            jax.ShapeDtypeStruct((B, 1, _KPAD), jnp.int32),
        ],
    )(sv.reshape(B, 1, 2 * _CBUF), si.reshape(B, 1, 2 * _CBUF))

    cand = _sc_gather()(packed.reshape(B * _NPAD, 8), gi.reshape(B, _KPAD))

    boxes, scores, labels, keep = pl.pallas_call(
        _nms_body,
        in_specs=[
            pl.BlockSpec((B, _KPAD, 8), lambda: (0, 0, 0)),
            pl.BlockSpec((B, 1, _KPAD), lambda: (0, 0, 0)),
        ],
        out_specs=[
            pl.BlockSpec((B, _TOP_K, 4), lambda: (0, 0, 0)),
            pl.BlockSpec((B, 1, _TOP_K), lambda: (0, 0, 0)),
            pl.BlockSpec((B, 1, _TOP_K), lambda: (0, 0, 0)),
            pl.BlockSpec((B, 1, _TOP_K), lambda: (0, 0, 0)),
        ],
        out_shape=[
            jax.ShapeDtypeStruct((B, _TOP_K, 4), jnp.float32),
            jax.ShapeDtypeStruct((B, 1, _TOP_K), jnp.float32),
            jax.ShapeDtypeStruct((B, 1, _TOP_K), jnp.int32),
            jax.ShapeDtypeStruct((B, 1, _TOP_K), jnp.int32),
        ],
        scratch_shapes=[pltpu.VMEM((_B, _KPAD, _KPAD), jnp.float32)],
    )(cand, gv)

    return (boxes,
            scores.reshape(B, _TOP_K),
            labels.reshape(B, _TOP_K),
            keep.reshape(B, _TOP_K).astype(bool))
